# baseline (speedup 1.0000x reference)
import jax
import jax.numpy as jnp
from jax.experimental import pallas as pl
from jax.experimental.pallas import tpu as pltpu

_CHUNK = 256
_NBUF = 12


def _router_body(x_hbm, w_ref, o_ref, buf, sems):
    M = x_hbm.shape[0]
    nchunks = M // _CHUNK
    w = w_ref[...]

    def _copy(chunk_idx, slot):
        return pltpu.make_async_copy(
            x_hbm.at[pl.ds(chunk_idx * _CHUNK, _CHUNK), :],
            buf.at[slot],
            sems.at[slot],
        )

    for s in range(_NBUF):
        _copy(s, s).start()

    def step(i, carry):
        slot = jax.lax.rem(i, _NBUF)
        _copy(i, slot).wait()
        x = buf[slot]
        logits = jax.lax.dot_general(
            x, w,
            dimension_numbers=(((1,), (1,)), ((), ())),
            preferred_element_type=jnp.float32,
        )
        m = jnp.max(logits, axis=-1, keepdims=True)
        e = jnp.exp(logits - m)
        o_ref[pl.ds(i * _CHUNK, _CHUNK), :] = e / jnp.sum(e, axis=-1, keepdims=True)

        @pl.when(i + _NBUF < nchunks)
        def _():
            _copy(i + _NBUF, slot).start()

        return carry

    jax.lax.fori_loop(0, nchunks, step, 0)


def kernel(inputs, W):
    M, K = inputs.shape
    E = W.shape[0]
    return pl.pallas_call(
        _router_body,
        in_specs=[
            pl.BlockSpec(memory_space=pltpu.MemorySpace.HBM),
            pl.BlockSpec((E, K), lambda: (0, 0)),
        ],
        out_specs=pl.BlockSpec((M, E), lambda: (0, 0)),
        out_shape=jax.ShapeDtypeStruct((M, E), jnp.float32),
        scratch_shapes=[
            pltpu.VMEM((_NBUF, _CHUNK, K), jnp.float32),
            pltpu.SemaphoreType.DMA((_NBUF,)),
        ],
    )(inputs, W)
